# baseline (speedup 1.0000x reference)
import functools

import jax
import jax.numpy as jnp
from jax import lax
from jax.experimental import pallas as pl
from jax.experimental.pallas import tpu as pltpu
from jax.experimental.pallas import tpu_sc as plsc

NC = 2
NS = 16
L = 16
NW = NC * NS
DIM = 64
CTX = 200
TOK = CTX + 1
HID = 128
IN_DIM = DIM * TOK + 1
TPW = 7
ROWS = TPW * DIM
PAD = 240


def _sc_body(category_h, title_h, emb_h, emb2_h, w1_h, out_h,
             title_v, cat_v, rows_v, w1_v, acc_v, sem_a, sem_b):
    c = lax.axis_index("c")
    s = lax.axis_index("s")
    wid = s * NC + c
    tb = jnp.minimum(TPW * wid, TOK - TPW)

    for q in range(PAD // L):
        title_v[pl.ds(q * L, L)] = jnp.zeros((L,), jnp.int32)
    pltpu.sync_copy(title_h, title_v.at[pl.ds(L, CTX)])
    tv = title_v[pl.ds(tb + (L - 1), L)]

    copies = [pltpu.async_copy(emb_h.at[pl.ds(tv[k], 1)],
                               rows_v.at[pl.ds(k, 1)], sem_a)
              for k in range(TPW)]
    for cp in copies:
        cp.wait()

    @pl.when(wid == 0)
    def _():
        pltpu.sync_copy(category_h, cat_v.at[pl.ds(0, 1)])
        cv = cat_v[pl.ds(0, L)]
        pltpu.async_copy(emb2_h.at[pl.ds(cv[0], 1)],
                         rows_v.at[pl.ds(0, 1)], sem_b).wait()

    pltpu.sync_copy(w1_h.at[pl.ds(DIM * tb, ROWS)], w1_v)

    hi = jnp.minimum(TPW * wid + TPW, TOK)
    for k in range(TPW):
        t = tb + k
        scale = jnp.where((t >= TPW * wid) & (t < hi), 1.0, 0.0).astype(jnp.float32)
        for q in range(DIM // L):
            rows_v[k, pl.ds(q * L, L)] = rows_v[k, pl.ds(q * L, L)] * scale

    def body(g, acc):
        k = g // (DIM // L)
        db = g - k * (DIM // L)
        xv = rows_v[k, pl.ds(db * L, L)]
        for e in range(L):
            xb = jnp.broadcast_to(xv[e], (L,))
            i = g * L + e
            acc = tuple(acc[j] + xb * w1_v[i, pl.ds(j * L, L)]
                        for j in range(HID // L))
        return acc

    acc0 = tuple(jnp.zeros((L,), jnp.float32) for _ in range(HID // L))
    acc = lax.fori_loop(0, ROWS // L, body, acc0)
    for j in range(HID // L):
        acc_v[0, pl.ds(j * L, L)] = acc[j]
    pltpu.sync_copy(acc_v, out_h.at[pl.ds(wid, 1)])


_sc_partials = functools.partial(
    pl.kernel,
    mesh=plsc.VectorSubcoreMesh(core_axis_name="c", subcore_axis_name="s"),
    out_type=jax.ShapeDtypeStruct((NW, HID), jnp.float32),
    scratch_types=[
        pltpu.VMEM((PAD,), jnp.int32),
        pltpu.VMEM((L,), jnp.int32),
        pltpu.VMEM((L, DIM), jnp.float32),
        pltpu.VMEM((ROWS, HID), jnp.float32),
        pltpu.VMEM((1, HID), jnp.float32),
        pltpu.SemaphoreType.DMA,
        pltpu.SemaphoreType.DMA,
    ],
)(_sc_body)


def _epilogue_body(p_ref, w1l_ref, b1_ref, q_ref, w2t_ref, b2_ref, o_ref):
    h = (jnp.sum(p_ref[...], axis=0, keepdims=True) + b1_ref[...]
         + q_ref[0, 0] * w1l_ref[...])
    h = jnp.maximum(h, 0.0)
    o = jnp.sum(h * w2t_ref[...], axis=1, keepdims=True) + b2_ref[...]
    o_ref[...] = 1.0 / (1.0 + jnp.exp(-o))


def kernel(category, title, quantity, emb, emb2, W1, b1, W2, b2):
    partials = _sc_partials(
        category.astype(jnp.int32), title.astype(jnp.int32), emb, emb2, W1)
    w1_last = lax.slice(W1, (IN_DIM - 1, 0), (IN_DIM, HID))
    return pl.pallas_call(
        _epilogue_body,
        out_shape=jax.ShapeDtypeStruct((1, 1), jnp.float32),
    )(partials, w1_last, b1.reshape(1, HID), quantity.reshape(1, 1),
      W2.reshape(1, HID), b2.reshape(1, 1))
